# baseline (speedup 1.0000x reference)
import jax
import jax.numpy as jnp
from jax import lax
from jax.experimental import pallas as pl
from jax.experimental.pallas import tpu as pltpu
from jax.experimental.pallas import tpu_sc as plsc

N_NODES = 10000
N_EDGES = 320000
D_IN = 128
D_HID = 256
D_OUT = 128
DH = D_IN // 2

NC = 2
NS = 16
L = 16

N_PAD = 10240
ROWS_PER_TILE = N_PAD // NS
CHUNK = 128
CHUNKS_PER_TILE = 80
E_PAD = NC * NS * CHUNKS_PER_TILE * CHUNK
DEG_PER_TILE = N_EDGES // NS

_mesh = plsc.VectorSubcoreMesh(core_axis_name="c", subcore_axis_name="s")
_sc_params = pltpu.CompilerParams(needs_layout_passes=False,
                                  use_tc_tiling_on_sc=False)


def _deg_dis_xs_body(dst_hbm, xlo_hbm, xhi_hbm, dis_hbm, xslo_hbm, xshi_hbm,
                     dst_v, deg_v, stage, slice_v, dis_v, blo, bhi):
    c = lax.axis_index("c")
    s = lax.axis_index("s")

    pltpu.sync_copy(dst_hbm.at[s], dst_v)

    def zero(i, _):
        deg_v[pl.ds(i * L, L)] = jnp.zeros((L,), jnp.float32)
        return 0
    lax.fori_loop(0, N_PAD // L, zero, 0)

    ones = jnp.ones((L,), jnp.float32)

    def count(i, _):
        idx = dst_v[pl.ds(i * L, L)]
        plsc.addupdate_scatter(deg_v, [idx], ones)
        return 0
    lax.fori_loop(0, DEG_PER_TILE // L, count, 0)

    pltpu.sync_copy(deg_v, stage.at[s])
    plsc.subcore_barrier()
    pltpu.sync_copy(stage.at[:, pl.ds(s * ROWS_PER_TILE, ROWS_PER_TILE)],
                    slice_v)

    def reduce_k(k, _):
        def add_row(r, a):
            return a + slice_v[r, pl.ds(k * L, L)]
        tot = lax.fori_loop(0, NS, add_row, jnp.zeros((L,), jnp.float32))
        d = tot + 1.0
        i32 = plsc.bitcast(d, jnp.int32)
        i32 = 0x5F3759DF - (i32 >> 1)
        y = plsc.bitcast(i32, jnp.float32)
        for _ in range(3):
            y = y * (1.5 - 0.5 * d * y * y)
        dis_v[pl.ds(k * L, L)] = y
        return 0
    lax.fori_loop(0, ROWS_PER_TILE // L, reduce_k, 0)

    @pl.when(c == 0)
    def _():
        pltpu.sync_copy(dis_v, dis_hbm.at[pl.ds(s * ROWS_PER_TILE,
                                                ROWS_PER_TILE)])

    half = ROWS_PER_TILE // NC
    base = s * ROWS_PER_TILE + c * half
    loc0 = c * half
    XB = 80

    def xs_chunk(ch, _):
        row0 = base + ch * XB
        pltpu.sync_copy(xlo_hbm.at[pl.ds(row0, XB)], blo)
        pltpu.sync_copy(xhi_hbm.at[pl.ds(row0, XB)], bhi)

        def grp_fn(g, _):
            dvs = dis_v[pl.ds(loc0 + ch * XB + g * L, L)]
            for r in range(L):
                dv = dvs[r]

                def col_fn(j, _, r=r, dv=dv):
                    row = g * L + r
                    blo[row, pl.ds(j * L, L)] = blo[row, pl.ds(j * L, L)] * dv
                    bhi[row, pl.ds(j * L, L)] = bhi[row, pl.ds(j * L, L)] * dv
                    return 0
                lax.fori_loop(0, DH // L, col_fn, 0)
            return 0
        lax.fori_loop(0, XB // L, grp_fn, 0)
        pltpu.sync_copy(blo, xslo_hbm.at[pl.ds(row0, XB)])
        pltpu.sync_copy(bhi, xshi_hbm.at[pl.ds(row0, XB)])
        return 0
    lax.fori_loop(0, half // XB, xs_chunk, 0)


_deg_dis_xs = pl.kernel(
    _deg_dis_xs_body,
    out_type=(jax.ShapeDtypeStruct((N_PAD,), jnp.float32),
              jax.ShapeDtypeStruct((N_PAD, DH), jnp.float32),
              jax.ShapeDtypeStruct((N_PAD, DH), jnp.float32)),
    mesh=_mesh,
    scratch_types=[
        pltpu.VMEM((DEG_PER_TILE,), jnp.int32),
        pltpu.VMEM((N_PAD,), jnp.float32),
        pltpu.VMEM_SHARED((NS, N_PAD), jnp.float32),
        pltpu.VMEM((NS, ROWS_PER_TILE), jnp.float32),
        pltpu.VMEM((ROWS_PER_TILE,), jnp.float32),
        pltpu.VMEM((80, DH), jnp.float32),
        pltpu.VMEM((80, DH), jnp.float32),
    ],
    compiler_params=_sc_params,
)


def _scatter_body(tlo_hbm, thi_hbm, src_hbm, dst_hbm, olo_hbm, ohi_hbm,
                  sidx, didx, buf0, buf1, zbuf, acc, sem0, sem1):
    c = lax.axis_index("c")
    s = lax.axis_index("s")
    wid = s * NC + c

    pltpu.sync_copy(src_hbm.at[wid], sidx)
    pltpu.sync_copy(dst_hbm.at[wid], didx)

    ZR = 320
    row0 = s * ROWS_PER_TILE

    def zero(i, _):
        r = i // (DH // L)
        j = i % (DH // L)
        zbuf[r, pl.ds(j * L, L)] = jnp.zeros((L,), jnp.float32)
        return 0
    lax.fori_loop(0, ZR * (DH // L), zero, 0)

    for tab_hbm, out_hbm in ((tlo_hbm, olo_hbm), (thi_hbm, ohi_hbm)):
        pltpu.sync_copy(zbuf, acc.at[pl.ds(row0, ZR)])
        pltpu.sync_copy(zbuf, acc.at[pl.ds(row0 + ZR, ZR)])
        plsc.subcore_barrier()

        pltpu.async_copy(tab_hbm.at[sidx.at[0]], buf0, sem0)
        pltpu.async_copy(tab_hbm.at[sidx.at[1]], buf1, sem1)

        def step(i, _, tab_hbm=tab_hbm):
            j = i * 2
            pltpu.make_async_copy(tab_hbm.at[sidx.at[j]], buf0, sem0).wait()
            pltpu.sync_copy(buf0, acc.at[didx.at[j]], add=True)
            pltpu.async_copy(tab_hbm.at[sidx.at[j + 2]], buf0, sem0)
            pltpu.make_async_copy(tab_hbm.at[sidx.at[j + 1]], buf1,
                                  sem1).wait()
            pltpu.sync_copy(buf1, acc.at[didx.at[j + 1]], add=True)
            pltpu.async_copy(tab_hbm.at[sidx.at[j + 3]], buf1, sem1)
            return 0
        lax.fori_loop(0, CHUNKS_PER_TILE // 2 - 1, step, 0)

        jl = CHUNKS_PER_TILE - 2
        pltpu.make_async_copy(tab_hbm.at[sidx.at[jl]], buf0, sem0).wait()
        pltpu.sync_copy(buf0, acc.at[didx.at[jl]], add=True)
        pltpu.make_async_copy(tab_hbm.at[sidx.at[jl + 1]], buf1, sem1).wait()
        pltpu.sync_copy(buf1, acc.at[didx.at[jl + 1]], add=True)

        plsc.subcore_barrier()
        pltpu.sync_copy(acc.at[pl.ds(row0, ROWS_PER_TILE)],
                        out_hbm.at[c, pl.ds(row0, ROWS_PER_TILE)])


_scatter = pl.kernel(
    _scatter_body,
    out_type=(jax.ShapeDtypeStruct((NC, N_PAD, DH), jnp.float32),
              jax.ShapeDtypeStruct((NC, N_PAD, DH), jnp.float32)),
    mesh=_mesh,
    scratch_types=[
        pltpu.VMEM((CHUNKS_PER_TILE, CHUNK), jnp.int32),
        pltpu.VMEM((CHUNKS_PER_TILE, CHUNK), jnp.int32),
        pltpu.VMEM((CHUNK, DH), jnp.float32),
        pltpu.VMEM((CHUNK, DH), jnp.float32),
        pltpu.VMEM((320, DH), jnp.float32),
        pltpu.VMEM_SHARED((N_PAD, DH), jnp.float32),
        pltpu.SemaphoreType.DMA,
        pltpu.SemaphoreType.DMA,
    ],
    compiler_params=_sc_params,
)


def _mm_body(plo_ref, phi_ref, xslo_ref, xshi_ref, dis_ref,
             w1_ref, b1_ref, w2_ref, glo_ref, ghi_ref):
    a_lo = (plo_ref[0] + plo_ref[1] + xslo_ref[...]) * dis_ref[...]
    a_hi = (phi_ref[0] + phi_ref[1] + xshi_ref[...]) * dis_ref[...]
    a = jnp.concatenate([a_lo, a_hi], axis=1)
    h = jnp.dot(a, w1_ref[...], preferred_element_type=jnp.float32)
    h = jnp.maximum(h + b1_ref[...], 0.0)
    g = jnp.dot(h, w2_ref[...], preferred_element_type=jnp.float32)
    g = g * dis_ref[...]
    glo_ref[...] = g[:, :DH]
    ghi_ref[...] = g[:, DH:]


def _final_body(plo_ref, phi_ref, glo_ref, ghi_ref, dis_ref, b2_ref,
                out_ref):
    o_lo = (plo_ref[0] + plo_ref[1] + glo_ref[...]) * dis_ref[...]
    o_hi = (phi_ref[0] + phi_ref[1] + ghi_ref[...]) * dis_ref[...]
    out_ref[...] = jnp.concatenate([o_lo, o_hi], axis=1) + b2_ref[...]


_RB = 640
_GRID = N_PAD // _RB

_mm = pl.pallas_call(
    _mm_body,
    grid=(_GRID,),
    in_specs=[
        pl.BlockSpec((NC, _RB, DH), lambda i: (0, i, 0)),
        pl.BlockSpec((NC, _RB, DH), lambda i: (0, i, 0)),
        pl.BlockSpec((_RB, DH), lambda i: (i, 0)),
        pl.BlockSpec((_RB, DH), lambda i: (i, 0)),
        pl.BlockSpec((_RB, 1), lambda i: (i, 0)),
        pl.BlockSpec((D_IN, D_HID), lambda i: (0, 0)),
        pl.BlockSpec((1, D_HID), lambda i: (0, 0)),
        pl.BlockSpec((D_HID, D_OUT), lambda i: (0, 0)),
    ],
    out_specs=[pl.BlockSpec((_RB, DH), lambda i: (i, 0)),
               pl.BlockSpec((_RB, DH), lambda i: (i, 0))],
    out_shape=[jax.ShapeDtypeStruct((N_PAD, DH), jnp.float32),
               jax.ShapeDtypeStruct((N_PAD, DH), jnp.float32)],
)

_final = pl.pallas_call(
    _final_body,
    grid=(_GRID,),
    in_specs=[
        pl.BlockSpec((NC, _RB, DH), lambda i: (0, i, 0)),
        pl.BlockSpec((NC, _RB, DH), lambda i: (0, i, 0)),
        pl.BlockSpec((_RB, DH), lambda i: (i, 0)),
        pl.BlockSpec((_RB, DH), lambda i: (i, 0)),
        pl.BlockSpec((_RB, 1), lambda i: (i, 0)),
        pl.BlockSpec((1, D_OUT), lambda i: (0, 0)),
    ],
    out_specs=pl.BlockSpec((_RB, D_OUT), lambda i: (i, 0)),
    out_shape=jax.ShapeDtypeStruct((N_PAD, D_OUT), jnp.float32),
)


@jax.jit
def _run(x, edge_index, W1, b1, W2, b2):
    src = edge_index[0].astype(jnp.int32)
    dst = edge_index[1].astype(jnp.int32)

    pad = E_PAD - N_EDGES
    src_p = jnp.concatenate([src, jnp.zeros((pad,), jnp.int32)])
    pad_dst = N_NODES + (jnp.arange(pad, dtype=jnp.int32) % (N_PAD - N_NODES))
    dst_p = jnp.concatenate([dst, pad_dst])
    src_p = src_p.reshape(NC * NS, CHUNKS_PER_TILE, CHUNK)
    dst_p = dst_p.reshape(NC * NS, CHUNKS_PER_TILE, CHUNK)
    dst_deg = dst.reshape(NS, DEG_PER_TILE)

    x_pad = jnp.zeros((N_PAD, D_IN), jnp.float32).at[:N_NODES].set(x)
    x_lo = x_pad[:, :DH]
    x_hi = x_pad[:, DH:]

    dis, xs_lo, xs_hi = _deg_dis_xs(dst_deg, x_lo, x_hi)
    dis_col = dis.reshape(N_PAD, 1)

    p1_lo, p1_hi = _scatter(xs_lo, xs_hi, src_p, dst_p)
    gs_lo, gs_hi = _mm(p1_lo, p1_hi, xs_lo, xs_hi, dis_col,
                       W1, b1.reshape(1, D_HID), W2)
    p2_lo, p2_hi = _scatter(gs_lo, gs_hi, src_p, dst_p)
    out = _final(p2_lo, p2_hi, gs_lo, gs_hi, dis_col, b2.reshape(1, D_OUT))
    return out[:N_NODES]


def kernel(x, edge_index, edge_attr, W1, b1, W2, b2):
    return _run(x, edge_index, W1, b1, W2, b2)


# speedup vs baseline: 2.7349x; 2.7349x over previous
import jax
import jax.numpy as jnp
from jax import lax
from jax.experimental import pallas as pl
from jax.experimental.pallas import tpu as pltpu
from jax.experimental.pallas import tpu_sc as plsc

N_NODES = 10000
N_EDGES = 320000
D_IN = 128
D_HID = 256
D_OUT = 128
DH = D_IN // 2

NC = 2
NS = 16
L = 16

N_PAD = 10240
ROWS_PER_TILE = N_PAD // NS
CHUNK = 128
CHUNKS_PER_TILE = 80
E_PAD = NC * NS * CHUNKS_PER_TILE * CHUNK
DEG_PER_TILE = N_EDGES // NS

_mesh = plsc.VectorSubcoreMesh(core_axis_name="c", subcore_axis_name="s")
_sc_params = pltpu.CompilerParams(needs_layout_passes=False,
                                  use_tc_tiling_on_sc=False)


def _deg_dis_xs_body(dst_hbm, xlo_hbm, xhi_hbm, dis_hbm, xslo_hbm, xshi_hbm,
                     dst_v, deg_v, stage, slice_v, dis_v, blo, bhi):
    c = lax.axis_index("c")
    s = lax.axis_index("s")

    pltpu.sync_copy(dst_hbm.at[s], dst_v)

    def zero(i, _):
        deg_v[pl.ds(i * L, L)] = jnp.zeros((L,), jnp.float32)
        return 0
    lax.fori_loop(0, N_PAD // L, zero, 0)

    ones = jnp.ones((L,), jnp.float32)

    def count(i, _):
        idx = dst_v[pl.ds(i * L, L)]
        plsc.addupdate_scatter(deg_v, [idx], ones)
        return 0
    lax.fori_loop(0, DEG_PER_TILE // L, count, 0)

    pltpu.sync_copy(deg_v, stage.at[s])
    plsc.subcore_barrier()
    pltpu.sync_copy(stage.at[:, pl.ds(s * ROWS_PER_TILE, ROWS_PER_TILE)],
                    slice_v)

    def reduce_k(k, _):
        def add_row(r, a):
            return a + slice_v[r, pl.ds(k * L, L)]
        tot = lax.fori_loop(0, NS, add_row, jnp.zeros((L,), jnp.float32))
        d = tot + 1.0
        i32 = plsc.bitcast(d, jnp.int32)
        i32 = 0x5F3759DF - (i32 >> 1)
        y = plsc.bitcast(i32, jnp.float32)
        for _ in range(3):
            y = y * (1.5 - 0.5 * d * y * y)
        dis_v[pl.ds(k * L, L)] = y
        return 0
    lax.fori_loop(0, ROWS_PER_TILE // L, reduce_k, 0)

    @pl.when(c == 0)
    def _():
        pltpu.sync_copy(dis_v, dis_hbm.at[pl.ds(s * ROWS_PER_TILE,
                                                ROWS_PER_TILE)])

    half = ROWS_PER_TILE // NC
    base = s * ROWS_PER_TILE + c * half
    loc0 = c * half
    XB = 80

    def xs_chunk(ch, _):
        row0 = base + ch * XB
        pltpu.sync_copy(xlo_hbm.at[pl.ds(row0, XB)], blo)
        pltpu.sync_copy(xhi_hbm.at[pl.ds(row0, XB)], bhi)

        def grp_fn(g, _):
            dvs = dis_v[pl.ds(loc0 + ch * XB + g * L, L)]
            for r in range(L):
                dv = dvs[r]

                def col_fn(j, _, r=r, dv=dv):
                    row = g * L + r
                    blo[row, pl.ds(j * L, L)] = blo[row, pl.ds(j * L, L)] * dv
                    bhi[row, pl.ds(j * L, L)] = bhi[row, pl.ds(j * L, L)] * dv
                    return 0
                lax.fori_loop(0, DH // L, col_fn, 0)
            return 0
        lax.fori_loop(0, XB // L, grp_fn, 0)
        pltpu.sync_copy(blo, xslo_hbm.at[pl.ds(row0, XB)])
        pltpu.sync_copy(bhi, xshi_hbm.at[pl.ds(row0, XB)])
        return 0
    lax.fori_loop(0, half // XB, xs_chunk, 0)


_deg_dis_xs = pl.kernel(
    _deg_dis_xs_body,
    out_type=(jax.ShapeDtypeStruct((N_PAD,), jnp.float32),
              jax.ShapeDtypeStruct((N_PAD, DH), jnp.float32),
              jax.ShapeDtypeStruct((N_PAD, DH), jnp.float32)),
    mesh=_mesh,
    scratch_types=[
        pltpu.VMEM((DEG_PER_TILE,), jnp.int32),
        pltpu.VMEM((N_PAD,), jnp.float32),
        pltpu.VMEM_SHARED((NS, N_PAD), jnp.float32),
        pltpu.VMEM((NS, ROWS_PER_TILE), jnp.float32),
        pltpu.VMEM((ROWS_PER_TILE,), jnp.float32),
        pltpu.VMEM((80, DH), jnp.float32),
        pltpu.VMEM((80, DH), jnp.float32),
    ],
    compiler_params=_sc_params,
)


def _scatter_body(tlo_hbm, thi_hbm, src_hbm, dst_hbm, olo_hbm, ohi_hbm,
                  sidx, didx, buf0, buf1, zbuf, acc, sem0, sem1):
    c = lax.axis_index("c")
    s = lax.axis_index("s")
    wid = s * NC + c

    pltpu.sync_copy(src_hbm.at[wid], sidx)
    pltpu.sync_copy(dst_hbm.at[wid], didx)

    ZR = 320
    row0 = s * ROWS_PER_TILE

    def zero(i, _):
        r = i // (DH // L)
        j = i % (DH // L)
        zbuf[r, pl.ds(j * L, L)] = jnp.zeros((L,), jnp.float32)
        return 0
    lax.fori_loop(0, ZR * (DH // L), zero, 0)

    for tab_hbm, out_hbm in ((tlo_hbm, olo_hbm), (thi_hbm, ohi_hbm)):
        pltpu.sync_copy(zbuf, acc.at[pl.ds(row0, ZR)])
        pltpu.sync_copy(zbuf, acc.at[pl.ds(row0 + ZR, ZR)])
        plsc.subcore_barrier()

        pltpu.async_copy(tab_hbm.at[sidx.at[0]], buf0, sem0)
        pltpu.async_copy(tab_hbm.at[sidx.at[1]], buf1, sem1)

        def step(i, _, tab_hbm=tab_hbm):
            j = i * 2
            pltpu.make_async_copy(tab_hbm.at[sidx.at[j]], buf0, sem0).wait()
            pltpu.sync_copy(buf0, acc.at[didx.at[j]], add=True)
            pltpu.async_copy(tab_hbm.at[sidx.at[j + 2]], buf0, sem0)
            pltpu.make_async_copy(tab_hbm.at[sidx.at[j + 1]], buf1,
                                  sem1).wait()
            pltpu.sync_copy(buf1, acc.at[didx.at[j + 1]], add=True)
            pltpu.async_copy(tab_hbm.at[sidx.at[j + 3]], buf1, sem1)
            return 0
        lax.fori_loop(0, CHUNKS_PER_TILE // 2 - 1, step, 0)

        jl = CHUNKS_PER_TILE - 2
        pltpu.make_async_copy(tab_hbm.at[sidx.at[jl]], buf0, sem0).wait()
        pltpu.sync_copy(buf0, acc.at[didx.at[jl]], add=True)
        pltpu.make_async_copy(tab_hbm.at[sidx.at[jl + 1]], buf1, sem1).wait()
        pltpu.sync_copy(buf1, acc.at[didx.at[jl + 1]], add=True)

        plsc.subcore_barrier()
        pltpu.sync_copy(acc.at[pl.ds(row0, ROWS_PER_TILE)],
                        out_hbm.at[c, pl.ds(row0, ROWS_PER_TILE)])


_scatter = pl.kernel(
    _scatter_body,
    out_type=(jax.ShapeDtypeStruct((NC, N_PAD, DH), jnp.float32),
              jax.ShapeDtypeStruct((NC, N_PAD, DH), jnp.float32)),
    mesh=_mesh,
    scratch_types=[
        pltpu.VMEM((CHUNKS_PER_TILE, CHUNK), jnp.int32),
        pltpu.VMEM((CHUNKS_PER_TILE, CHUNK), jnp.int32),
        pltpu.VMEM((CHUNK, DH), jnp.float32),
        pltpu.VMEM((CHUNK, DH), jnp.float32),
        pltpu.VMEM((320, DH), jnp.float32),
        pltpu.VMEM_SHARED((N_PAD, DH), jnp.float32),
        pltpu.SemaphoreType.DMA,
        pltpu.SemaphoreType.DMA,
    ],
    compiler_params=_sc_params,
)


def _mm_body(plo_ref, phi_ref, xslo_ref, xshi_ref, dis_ref,
             w1_ref, b1_ref, w2_ref, glo_ref, ghi_ref):
    a_lo = (plo_ref[0] + plo_ref[1] + xslo_ref[...]) * dis_ref[...]
    a_hi = (phi_ref[0] + phi_ref[1] + xshi_ref[...]) * dis_ref[...]
    a = jnp.concatenate([a_lo, a_hi], axis=1)
    h = jnp.dot(a, w1_ref[...], preferred_element_type=jnp.float32)
    h = jnp.maximum(h + b1_ref[...], 0.0)
    g = jnp.dot(h, w2_ref[...], preferred_element_type=jnp.float32)
    g = g * dis_ref[...]
    glo_ref[...] = g[:, :DH]
    ghi_ref[...] = g[:, DH:]


def _final_body(plo_ref, phi_ref, glo_ref, ghi_ref, dis_ref, b2_ref,
                out_ref):
    o_lo = (plo_ref[0] + plo_ref[1] + glo_ref[...]) * dis_ref[...]
    o_hi = (phi_ref[0] + phi_ref[1] + ghi_ref[...]) * dis_ref[...]
    out_ref[...] = jnp.concatenate([o_lo, o_hi], axis=1) + b2_ref[...]


_RB = 640
_GRID = N_PAD // _RB

_mm = pl.pallas_call(
    _mm_body,
    grid=(_GRID,),
    in_specs=[
        pl.BlockSpec((NC, _RB, DH), lambda i: (0, i, 0)),
        pl.BlockSpec((NC, _RB, DH), lambda i: (0, i, 0)),
        pl.BlockSpec((_RB, DH), lambda i: (i, 0)),
        pl.BlockSpec((_RB, DH), lambda i: (i, 0)),
        pl.BlockSpec((_RB, 1), lambda i: (i, 0)),
        pl.BlockSpec((D_IN, D_HID), lambda i: (0, 0)),
        pl.BlockSpec((1, D_HID), lambda i: (0, 0)),
        pl.BlockSpec((D_HID, D_OUT), lambda i: (0, 0)),
    ],
    out_specs=[pl.BlockSpec((_RB, DH), lambda i: (i, 0)),
               pl.BlockSpec((_RB, DH), lambda i: (i, 0))],
    out_shape=[jax.ShapeDtypeStruct((N_PAD, DH), jnp.float32),
               jax.ShapeDtypeStruct((N_PAD, DH), jnp.float32)],
)

_final = pl.pallas_call(
    _final_body,
    grid=(_GRID,),
    in_specs=[
        pl.BlockSpec((NC, _RB, DH), lambda i: (0, i, 0)),
        pl.BlockSpec((NC, _RB, DH), lambda i: (0, i, 0)),
        pl.BlockSpec((_RB, DH), lambda i: (i, 0)),
        pl.BlockSpec((_RB, DH), lambda i: (i, 0)),
        pl.BlockSpec((_RB, 1), lambda i: (i, 0)),
        pl.BlockSpec((1, D_OUT), lambda i: (0, 0)),
    ],
    out_specs=pl.BlockSpec((_RB, D_OUT), lambda i: (i, 0)),
    out_shape=jax.ShapeDtypeStruct((N_PAD, D_OUT), jnp.float32),
)


@jax.jit
def _run(x, edge_index, W1, b1, W2, b2):
    src = edge_index[0].astype(jnp.int32)
    dst = edge_index[1].astype(jnp.int32)

    pad = E_PAD - N_EDGES
    pad_src = jnp.arange(pad, dtype=jnp.int32) % N_NODES
    pad_dst = N_NODES + (jnp.arange(pad, dtype=jnp.int32) % (N_PAD - N_NODES))
    src_p = jnp.concatenate([src, pad_src])
    dst_p = jnp.concatenate([dst, pad_dst])
    src_p = src_p.reshape(NC * NS, CHUNKS_PER_TILE, CHUNK)
    dst_p = dst_p.reshape(NC * NS, CHUNKS_PER_TILE, CHUNK)
    dst_deg = dst.reshape(NS, DEG_PER_TILE)

    x_pad = jnp.zeros((N_PAD, D_IN), jnp.float32).at[:N_NODES].set(x)
    x_lo = x_pad[:, :DH]
    x_hi = x_pad[:, DH:]

    dis, xs_lo, xs_hi = _deg_dis_xs(dst_deg, x_lo, x_hi)
    dis_col = dis.reshape(N_PAD, 1)

    p1_lo, p1_hi = _scatter(xs_lo, xs_hi, src_p, dst_p)
    gs_lo, gs_hi = _mm(p1_lo, p1_hi, xs_lo, xs_hi, dis_col,
                       W1, b1.reshape(1, D_HID), W2)
    p2_lo, p2_hi = _scatter(gs_lo, gs_hi, src_p, dst_p)
    out = _final(p2_lo, p2_hi, gs_lo, gs_hi, dis_col, b2.reshape(1, D_OUT))
    return out[:N_NODES]


def kernel(x, edge_index, edge_attr, W1, b1, W2, b2):
    return _run(x, edge_index, W1, b1, W2, b2)


# speedup vs baseline: 2.9200x; 1.0677x over previous
import jax
import jax.numpy as jnp
from jax import lax
from jax.experimental import pallas as pl
from jax.experimental.pallas import tpu as pltpu
from jax.experimental.pallas import tpu_sc as plsc

N_NODES = 10000
N_EDGES = 320000
D_IN = 128
D_HID = 256
D_OUT = 128
DH = D_IN // 2

NC = 2
NS = 16
L = 16

N_PAD = 10240
ROWS_PER_TILE = N_PAD // NS
CHUNK = 128
CHUNKS_PER_TILE = 80
E_PAD = NC * NS * CHUNKS_PER_TILE * CHUNK
DEG_PER_TILE = N_EDGES // NS

_mesh = plsc.VectorSubcoreMesh(core_axis_name="c", subcore_axis_name="s")
_sc_params = pltpu.CompilerParams(needs_layout_passes=False,
                                  use_tc_tiling_on_sc=False)


def _deg_dis_xs_body(dst_hbm, xlo_hbm, xhi_hbm, dis_hbm, xslo_hbm, xshi_hbm,
                     dst_v, deg_v, stage, slice_v, dis_v, blo, bhi):
    c = lax.axis_index("c")
    s = lax.axis_index("s")

    pltpu.sync_copy(dst_hbm.at[s], dst_v)

    def zero(i, _):
        deg_v[pl.ds(i * L, L)] = jnp.zeros((L,), jnp.float32)
        return 0
    lax.fori_loop(0, N_PAD // L, zero, 0)

    ones = jnp.ones((L,), jnp.float32)

    def count(i, _):
        idx = dst_v[pl.ds(i * L, L)]
        plsc.addupdate_scatter(deg_v, [idx], ones)
        return 0
    lax.fori_loop(0, DEG_PER_TILE // L, count, 0)

    pltpu.sync_copy(deg_v, stage.at[s])
    plsc.subcore_barrier()
    pltpu.sync_copy(stage.at[:, pl.ds(s * ROWS_PER_TILE, ROWS_PER_TILE)],
                    slice_v)

    def reduce_k(k, _):
        def add_row(r, a):
            return a + slice_v[r, pl.ds(k * L, L)]
        tot = lax.fori_loop(0, NS, add_row, jnp.zeros((L,), jnp.float32))
        d = tot + 1.0
        i32 = plsc.bitcast(d, jnp.int32)
        i32 = 0x5F3759DF - (i32 >> 1)
        y = plsc.bitcast(i32, jnp.float32)
        for _ in range(3):
            y = y * (1.5 - 0.5 * d * y * y)
        dis_v[pl.ds(k * L, L)] = y
        return 0
    lax.fori_loop(0, ROWS_PER_TILE // L, reduce_k, 0)

    @pl.when(c == 0)
    def _():
        pltpu.sync_copy(dis_v, dis_hbm.at[pl.ds(s * ROWS_PER_TILE,
                                                ROWS_PER_TILE)])

    half = ROWS_PER_TILE // NC
    base = s * ROWS_PER_TILE + c * half
    loc0 = c * half
    XB = 80

    def xs_chunk(ch, _):
        row0 = base + ch * XB
        pltpu.sync_copy(xlo_hbm.at[pl.ds(row0, XB)], blo)
        pltpu.sync_copy(xhi_hbm.at[pl.ds(row0, XB)], bhi)

        def grp_fn(g, _):
            dvs = dis_v[pl.ds(loc0 + ch * XB + g * L, L)]
            for r in range(L):
                dv = dvs[r]

                def col_fn(j, _, r=r, dv=dv):
                    row = g * L + r
                    blo[row, pl.ds(j * L, L)] = blo[row, pl.ds(j * L, L)] * dv
                    bhi[row, pl.ds(j * L, L)] = bhi[row, pl.ds(j * L, L)] * dv
                    return 0
                lax.fori_loop(0, DH // L, col_fn, 0)
            return 0
        lax.fori_loop(0, XB // L, grp_fn, 0)
        pltpu.sync_copy(blo, xslo_hbm.at[pl.ds(row0, XB)])
        pltpu.sync_copy(bhi, xshi_hbm.at[pl.ds(row0, XB)])
        return 0
    lax.fori_loop(0, half // XB, xs_chunk, 0)


_deg_dis_xs = pl.kernel(
    _deg_dis_xs_body,
    out_type=(jax.ShapeDtypeStruct((N_PAD,), jnp.float32),
              jax.ShapeDtypeStruct((N_PAD, DH), jnp.float32),
              jax.ShapeDtypeStruct((N_PAD, DH), jnp.float32)),
    mesh=_mesh,
    scratch_types=[
        pltpu.VMEM((DEG_PER_TILE,), jnp.int32),
        pltpu.VMEM((N_PAD,), jnp.float32),
        pltpu.VMEM_SHARED((NS, N_PAD), jnp.float32),
        pltpu.VMEM((NS, ROWS_PER_TILE), jnp.float32),
        pltpu.VMEM((ROWS_PER_TILE,), jnp.float32),
        pltpu.VMEM((80, DH), jnp.float32),
        pltpu.VMEM((80, DH), jnp.float32),
    ],
    compiler_params=_sc_params,
)


NBUF = 4


def _scatter_body(tlo_hbm, thi_hbm, src_hbm, dst_hbm, olo_hbm, ohi_hbm,
                  sidx, didx, bufs, zbuf, acc, gsems, ssems):
    c = lax.axis_index("c")
    s = lax.axis_index("s")
    wid = s * NC + c

    pltpu.sync_copy(src_hbm.at[wid], sidx)
    pltpu.sync_copy(dst_hbm.at[wid], didx)

    ZR = 320
    row0 = s * ROWS_PER_TILE

    def zero(i, _):
        r = i // (DH // L)
        j = i % (DH // L)
        zbuf[r, pl.ds(j * L, L)] = jnp.zeros((L,), jnp.float32)
        return 0
    lax.fori_loop(0, ZR * (DH // L), zero, 0)

    for tab_hbm, out_hbm in ((tlo_hbm, olo_hbm), (thi_hbm, ohi_hbm)):
        pltpu.sync_copy(zbuf, acc.at[pl.ds(row0, ZR)])
        pltpu.sync_copy(zbuf, acc.at[pl.ds(row0 + ZR, ZR)])
        plsc.subcore_barrier()

        for b in range(NBUF):
            pltpu.async_copy(tab_hbm.at[sidx.at[b]], bufs.at[b], gsems.at[b])

        def step(i, _, tab_hbm=tab_hbm):
            j0 = i * NBUF
            for b in range(NBUF):
                pltpu.make_async_copy(tab_hbm.at[sidx.at[j0 + b]],
                                      bufs.at[b], gsems.at[b]).wait()
                pltpu.async_copy(bufs.at[b], acc.at[didx.at[j0 + b]],
                                 ssems.at[b], add=True)
            for b in range(NBUF):
                pltpu.make_async_copy(bufs.at[b], acc.at[didx.at[j0 + b]],
                                      ssems.at[b]).wait()
                pltpu.async_copy(tab_hbm.at[sidx.at[j0 + b + NBUF]],
                                 bufs.at[b], gsems.at[b])
            return 0
        lax.fori_loop(0, CHUNKS_PER_TILE // NBUF - 1, step, 0)

        j0 = CHUNKS_PER_TILE - NBUF
        for b in range(NBUF):
            pltpu.make_async_copy(tab_hbm.at[sidx.at[j0 + b]],
                                  bufs.at[b], gsems.at[b]).wait()
            pltpu.async_copy(bufs.at[b], acc.at[didx.at[j0 + b]],
                             ssems.at[b], add=True)
        for b in range(NBUF):
            pltpu.make_async_copy(bufs.at[b], acc.at[didx.at[j0 + b]],
                                  ssems.at[b]).wait()

        plsc.subcore_barrier()
        pltpu.sync_copy(acc.at[pl.ds(row0, ROWS_PER_TILE)],
                        out_hbm.at[c, pl.ds(row0, ROWS_PER_TILE)])


_scatter = pl.kernel(
    _scatter_body,
    out_type=(jax.ShapeDtypeStruct((NC, N_PAD, DH), jnp.float32),
              jax.ShapeDtypeStruct((NC, N_PAD, DH), jnp.float32)),
    mesh=_mesh,
    scratch_types=[
        pltpu.VMEM((CHUNKS_PER_TILE, CHUNK), jnp.int32),
        pltpu.VMEM((CHUNKS_PER_TILE, CHUNK), jnp.int32),
        pltpu.VMEM((NBUF, CHUNK, DH), jnp.float32),
        pltpu.VMEM((320, DH), jnp.float32),
        pltpu.VMEM_SHARED((N_PAD, DH), jnp.float32),
        pltpu.SemaphoreType.DMA((NBUF,)),
        pltpu.SemaphoreType.DMA((NBUF,)),
    ],
    compiler_params=_sc_params,
)


def _mm_body(plo_ref, phi_ref, xslo_ref, xshi_ref, dis_ref,
             w1_ref, b1_ref, w2_ref, glo_ref, ghi_ref):
    a_lo = (plo_ref[0] + plo_ref[1] + xslo_ref[...]) * dis_ref[...]
    a_hi = (phi_ref[0] + phi_ref[1] + xshi_ref[...]) * dis_ref[...]
    a = jnp.concatenate([a_lo, a_hi], axis=1)
    h = jnp.dot(a, w1_ref[...], preferred_element_type=jnp.float32)
    h = jnp.maximum(h + b1_ref[...], 0.0)
    g = jnp.dot(h, w2_ref[...], preferred_element_type=jnp.float32)
    g = g * dis_ref[...]
    glo_ref[...] = g[:, :DH]
    ghi_ref[...] = g[:, DH:]


def _final_body(plo_ref, phi_ref, glo_ref, ghi_ref, dis_ref, b2_ref,
                out_ref):
    o_lo = (plo_ref[0] + plo_ref[1] + glo_ref[...]) * dis_ref[...]
    o_hi = (phi_ref[0] + phi_ref[1] + ghi_ref[...]) * dis_ref[...]
    out_ref[...] = jnp.concatenate([o_lo, o_hi], axis=1) + b2_ref[...]


_RB = 640
_GRID = N_PAD // _RB

_mm = pl.pallas_call(
    _mm_body,
    grid=(_GRID,),
    in_specs=[
        pl.BlockSpec((NC, _RB, DH), lambda i: (0, i, 0)),
        pl.BlockSpec((NC, _RB, DH), lambda i: (0, i, 0)),
        pl.BlockSpec((_RB, DH), lambda i: (i, 0)),
        pl.BlockSpec((_RB, DH), lambda i: (i, 0)),
        pl.BlockSpec((_RB, 1), lambda i: (i, 0)),
        pl.BlockSpec((D_IN, D_HID), lambda i: (0, 0)),
        pl.BlockSpec((1, D_HID), lambda i: (0, 0)),
        pl.BlockSpec((D_HID, D_OUT), lambda i: (0, 0)),
    ],
    out_specs=[pl.BlockSpec((_RB, DH), lambda i: (i, 0)),
               pl.BlockSpec((_RB, DH), lambda i: (i, 0))],
    out_shape=[jax.ShapeDtypeStruct((N_PAD, DH), jnp.float32),
               jax.ShapeDtypeStruct((N_PAD, DH), jnp.float32)],
)

_final = pl.pallas_call(
    _final_body,
    grid=(_GRID,),
    in_specs=[
        pl.BlockSpec((NC, _RB, DH), lambda i: (0, i, 0)),
        pl.BlockSpec((NC, _RB, DH), lambda i: (0, i, 0)),
        pl.BlockSpec((_RB, DH), lambda i: (i, 0)),
        pl.BlockSpec((_RB, DH), lambda i: (i, 0)),
        pl.BlockSpec((_RB, 1), lambda i: (i, 0)),
        pl.BlockSpec((1, D_OUT), lambda i: (0, 0)),
    ],
    out_specs=pl.BlockSpec((_RB, D_OUT), lambda i: (i, 0)),
    out_shape=jax.ShapeDtypeStruct((N_PAD, D_OUT), jnp.float32),
)


@jax.jit
def _run(x, edge_index, W1, b1, W2, b2):
    src = edge_index[0].astype(jnp.int32)
    dst = edge_index[1].astype(jnp.int32)

    pad = E_PAD - N_EDGES
    pad_src = jnp.arange(pad, dtype=jnp.int32) % N_NODES
    pad_dst = N_NODES + (jnp.arange(pad, dtype=jnp.int32) % (N_PAD - N_NODES))
    src_p = jnp.concatenate([src, pad_src])
    dst_p = jnp.concatenate([dst, pad_dst])
    src_p = src_p.reshape(NC * NS, CHUNKS_PER_TILE, CHUNK)
    dst_p = dst_p.reshape(NC * NS, CHUNKS_PER_TILE, CHUNK)
    dst_deg = dst.reshape(NS, DEG_PER_TILE)

    x_pad = jnp.zeros((N_PAD, D_IN), jnp.float32).at[:N_NODES].set(x)
    x_lo = x_pad[:, :DH]
    x_hi = x_pad[:, DH:]

    dis, xs_lo, xs_hi = _deg_dis_xs(dst_deg, x_lo, x_hi)
    dis_col = dis.reshape(N_PAD, 1)

    p1_lo, p1_hi = _scatter(xs_lo, xs_hi, src_p, dst_p)
    gs_lo, gs_hi = _mm(p1_lo, p1_hi, xs_lo, xs_hi, dis_col,
                       W1, b1.reshape(1, D_HID), W2)
    p2_lo, p2_hi = _scatter(gs_lo, gs_hi, src_p, dst_p)
    out = _final(p2_lo, p2_hi, gs_lo, gs_hi, dis_col, b2.reshape(1, D_OUT))
    return out[:N_NODES]


def kernel(x, edge_index, edge_attr, W1, b1, W2, b2):
    return _run(x, edge_index, W1, b1, W2, b2)


# speedup vs baseline: 3.0019x; 1.0281x over previous
import jax
import jax.numpy as jnp
from jax import lax
from jax.experimental import pallas as pl
from jax.experimental.pallas import tpu as pltpu
from jax.experimental.pallas import tpu_sc as plsc

N_NODES = 10000
N_EDGES = 320000
D_IN = 128
D_HID = 256
D_OUT = 128
DH = D_IN // 2

NC = 2
NS = 16
L = 16

N_PAD = 10240
ROWS_PER_TILE = N_PAD // NS
CHUNK = 128
CHUNKS_PER_TILE = 80
EROWS = N_EDGES // CHUNK
MAIN_ROWS = 78
TAIL_ROW0 = NC * NS * MAIN_ROWS
DROWS = EROWS // NS

_mesh = plsc.VectorSubcoreMesh(core_axis_name="c", subcore_axis_name="s")
_sc_params = pltpu.CompilerParams(needs_layout_passes=False,
                                  use_tc_tiling_on_sc=False)


def _deg_dis_xs_body(e3_hbm, x_hbm, dis_hbm, xslo_hbm, xshi_hbm,
                     dst_v, deg_v, stage, slice_v, dis_v, blo, bhi):
    c = lax.axis_index("c")
    s = lax.axis_index("s")

    pltpu.sync_copy(e3_hbm.at[1, pl.ds(s * DROWS, DROWS)],
                    dst_v.at[pl.ds(0, DROWS)])

    @pl.when(s < 4)
    def _():
        pltpu.sync_copy(e3_hbm.at[1, TAIL_ROW0 + s], dst_v.at[DROWS])

    def zero(i, _):
        deg_v[pl.ds(i * L, L)] = jnp.zeros((L,), jnp.float32)
        return 0
    lax.fori_loop(0, N_PAD // L, zero, 0)

    ones = jnp.ones((L,), jnp.float32)

    def count(i, _):
        idx = dst_v[i // (CHUNK // L), pl.ds((i % (CHUNK // L)) * L, L)]
        plsc.addupdate_scatter(deg_v, [idx], ones)
        return 0
    lax.fori_loop(0, DROWS * (CHUNK // L), count, 0)

    @pl.when(s < 4)
    def _():
        def count_tail(i, _):
            idx = dst_v[DROWS, pl.ds(i * L, L)]
            plsc.addupdate_scatter(deg_v, [idx], ones)
            return 0
        lax.fori_loop(0, CHUNK // L, count_tail, 0)

    pltpu.sync_copy(deg_v, stage.at[s])
    plsc.subcore_barrier()
    pltpu.sync_copy(stage.at[:, pl.ds(s * ROWS_PER_TILE, ROWS_PER_TILE)],
                    slice_v)

    def reduce_k(k, _):
        def add_row(r, a):
            return a + slice_v[r, pl.ds(k * L, L)]
        tot = lax.fori_loop(0, NS, add_row, jnp.zeros((L,), jnp.float32))
        d = tot + 1.0
        i32 = plsc.bitcast(d, jnp.int32)
        i32 = 0x5F3759DF - (i32 >> 1)
        y = plsc.bitcast(i32, jnp.float32)
        for _ in range(3):
            y = y * (1.5 - 0.5 * d * y * y)
        dis_v[pl.ds(k * L, L)] = y
        return 0
    lax.fori_loop(0, ROWS_PER_TILE // L, reduce_k, 0)

    @pl.when(c == 0)
    def _():
        pltpu.sync_copy(dis_v, dis_hbm.at[pl.ds(s * ROWS_PER_TILE,
                                                ROWS_PER_TILE)])

    half = ROWS_PER_TILE // NC
    base = s * ROWS_PER_TILE + c * half
    loc0 = c * half
    XB = 80

    def xs_chunk(ch, _):
        row0 = base + ch * XB

        @pl.when(row0 < N_NODES)
        def _():
            pltpu.sync_copy(x_hbm.at[pl.ds(row0, XB), pl.ds(0, DH)], blo)
            pltpu.sync_copy(x_hbm.at[pl.ds(row0, XB), pl.ds(DH, DH)], bhi)

            def grp_fn(g, _):
                dvs = dis_v[pl.ds(loc0 + ch * XB + g * L, L)]
                for r in range(L):
                    dv = dvs[r]

                    def col_fn(j, _, r=r, dv=dv):
                        row = g * L + r
                        blo[row, pl.ds(j * L, L)] = (
                            blo[row, pl.ds(j * L, L)] * dv)
                        bhi[row, pl.ds(j * L, L)] = (
                            bhi[row, pl.ds(j * L, L)] * dv)
                        return 0
                    lax.fori_loop(0, DH // L, col_fn, 0)
                return 0
            lax.fori_loop(0, XB // L, grp_fn, 0)
            pltpu.sync_copy(blo, xslo_hbm.at[pl.ds(row0, XB)])
            pltpu.sync_copy(bhi, xshi_hbm.at[pl.ds(row0, XB)])
        return 0
    lax.fori_loop(0, half // XB, xs_chunk, 0)


_deg_dis_xs = pl.kernel(
    _deg_dis_xs_body,
    out_type=(jax.ShapeDtypeStruct((N_PAD,), jnp.float32),
              jax.ShapeDtypeStruct((N_NODES, DH), jnp.float32),
              jax.ShapeDtypeStruct((N_NODES, DH), jnp.float32)),
    mesh=_mesh,
    scratch_types=[
        pltpu.VMEM((DROWS + 1, CHUNK), jnp.int32),
        pltpu.VMEM((N_PAD,), jnp.float32),
        pltpu.VMEM_SHARED((NS, N_PAD), jnp.float32),
        pltpu.VMEM((NS, ROWS_PER_TILE), jnp.float32),
        pltpu.VMEM((ROWS_PER_TILE,), jnp.float32),
        pltpu.VMEM((80, DH), jnp.float32),
        pltpu.VMEM((80, DH), jnp.float32),
    ],
    compiler_params=_sc_params,
)


NBUF = 4


def _scatter_body(tlo_hbm, thi_hbm, e3_hbm, olo_hbm, ohi_hbm,
                  sidx, didx, bufs, zbuf, acc, gsems, ssems):
    c = lax.axis_index("c")
    s = lax.axis_index("s")
    wid = s * NC + c

    pltpu.sync_copy(e3_hbm.at[0, pl.ds(wid * MAIN_ROWS, MAIN_ROWS)],
                    sidx.at[pl.ds(0, MAIN_ROWS)])
    pltpu.sync_copy(e3_hbm.at[1, pl.ds(wid * MAIN_ROWS, MAIN_ROWS)],
                    didx.at[pl.ds(0, MAIN_ROWS)])
    trow = TAIL_ROW0 + wid // 8
    tcol = (wid % 8) * L
    pltpu.sync_copy(e3_hbm.at[0, trow, pl.ds(tcol, L)],
                    sidx.at[MAIN_ROWS, pl.ds(0, L)])
    pltpu.sync_copy(e3_hbm.at[1, trow, pl.ds(tcol, L)],
                    didx.at[MAIN_ROWS, pl.ds(0, L)])
    iota = lax.iota(jnp.int32, L)
    pad_src = iota * 8 + wid
    for k in range(1, CHUNK // L):
        sidx[MAIN_ROWS, pl.ds(k * L, L)] = pad_src
        didx[MAIN_ROWS, pl.ds(k * L, L)] = (
            N_NODES + iota * 8 + (wid + k) % 8)
    for k in range(CHUNK // L):
        sidx[MAIN_ROWS + 1, pl.ds(k * L, L)] = pad_src
        didx[MAIN_ROWS + 1, pl.ds(k * L, L)] = (
            N_NODES + 128 + iota * 8 + (wid + k) % 8)

    ZR = 320
    row0 = s * ROWS_PER_TILE

    def zero(i, _):
        r = i // (DH // L)
        j = i % (DH // L)
        zbuf[r, pl.ds(j * L, L)] = jnp.zeros((L,), jnp.float32)
        return 0
    lax.fori_loop(0, ZR * (DH // L), zero, 0)

    for tab_hbm, out_hbm in ((tlo_hbm, olo_hbm), (thi_hbm, ohi_hbm)):
        pltpu.sync_copy(zbuf, acc.at[pl.ds(row0, ZR)])
        pltpu.sync_copy(zbuf, acc.at[pl.ds(row0 + ZR, ZR)])
        plsc.subcore_barrier()

        for b in range(NBUF):
            pltpu.async_copy(tab_hbm.at[sidx.at[b]], bufs.at[b], gsems.at[b])

        def step(i, _, tab_hbm=tab_hbm):
            j0 = i * NBUF
            for b in range(NBUF):
                pltpu.make_async_copy(tab_hbm.at[sidx.at[j0 + b]],
                                      bufs.at[b], gsems.at[b]).wait()
                pltpu.async_copy(bufs.at[b], acc.at[didx.at[j0 + b]],
                                 ssems.at[b], add=True)
            for b in range(NBUF):
                pltpu.make_async_copy(bufs.at[b], acc.at[didx.at[j0 + b]],
                                      ssems.at[b]).wait()
                pltpu.async_copy(tab_hbm.at[sidx.at[j0 + b + NBUF]],
                                 bufs.at[b], gsems.at[b])
            return 0
        lax.fori_loop(0, CHUNKS_PER_TILE // NBUF - 1, step, 0)

        j0 = CHUNKS_PER_TILE - NBUF
        for b in range(NBUF):
            pltpu.make_async_copy(tab_hbm.at[sidx.at[j0 + b]],
                                  bufs.at[b], gsems.at[b]).wait()
            pltpu.async_copy(bufs.at[b], acc.at[didx.at[j0 + b]],
                             ssems.at[b], add=True)
        for b in range(NBUF):
            pltpu.make_async_copy(bufs.at[b], acc.at[didx.at[j0 + b]],
                                  ssems.at[b]).wait()

        plsc.subcore_barrier()
        pltpu.sync_copy(acc.at[pl.ds(row0, ROWS_PER_TILE)],
                        out_hbm.at[c, pl.ds(row0, ROWS_PER_TILE)])


_scatter = pl.kernel(
    _scatter_body,
    out_type=(jax.ShapeDtypeStruct((NC, N_PAD, DH), jnp.float32),
              jax.ShapeDtypeStruct((NC, N_PAD, DH), jnp.float32)),
    mesh=_mesh,
    scratch_types=[
        pltpu.VMEM((CHUNKS_PER_TILE, CHUNK), jnp.int32),
        pltpu.VMEM((CHUNKS_PER_TILE, CHUNK), jnp.int32),
        pltpu.VMEM((NBUF, CHUNK, DH), jnp.float32),
        pltpu.VMEM((320, DH), jnp.float32),
        pltpu.VMEM_SHARED((N_PAD, DH), jnp.float32),
        pltpu.SemaphoreType.DMA((NBUF,)),
        pltpu.SemaphoreType.DMA((NBUF,)),
    ],
    compiler_params=_sc_params,
)


def _mm_body(plo_ref, phi_ref, xslo_ref, xshi_ref, dis_ref,
             w1_ref, b1_ref, w2_ref, glo_ref, ghi_ref):
    a_lo = (plo_ref[0] + plo_ref[1] + xslo_ref[...]) * dis_ref[...]
    a_hi = (phi_ref[0] + phi_ref[1] + xshi_ref[...]) * dis_ref[...]
    a = jnp.concatenate([a_lo, a_hi], axis=1)
    h = jnp.dot(a, w1_ref[...], preferred_element_type=jnp.float32)
    h = jnp.maximum(h + b1_ref[...], 0.0)
    g = jnp.dot(h, w2_ref[...], preferred_element_type=jnp.float32)
    g = g * dis_ref[...]
    glo_ref[...] = g[:, :DH]
    ghi_ref[...] = g[:, DH:]


def _final_body(plo_ref, phi_ref, glo_ref, ghi_ref, dis_ref, b2_ref,
                out_ref):
    o_lo = (plo_ref[0] + plo_ref[1] + glo_ref[...]) * dis_ref[...]
    o_hi = (phi_ref[0] + phi_ref[1] + ghi_ref[...]) * dis_ref[...]
    out_ref[...] = jnp.concatenate([o_lo, o_hi], axis=1) + b2_ref[...]


_RB = 400
_GRID = N_NODES // _RB

_mm = pl.pallas_call(
    _mm_body,
    grid=(_GRID,),
    in_specs=[
        pl.BlockSpec((NC, _RB, DH), lambda i: (0, i, 0)),
        pl.BlockSpec((NC, _RB, DH), lambda i: (0, i, 0)),
        pl.BlockSpec((_RB, DH), lambda i: (i, 0)),
        pl.BlockSpec((_RB, DH), lambda i: (i, 0)),
        pl.BlockSpec((_RB, 1), lambda i: (i, 0)),
        pl.BlockSpec((D_IN, D_HID), lambda i: (0, 0)),
        pl.BlockSpec((1, D_HID), lambda i: (0, 0)),
        pl.BlockSpec((D_HID, D_OUT), lambda i: (0, 0)),
    ],
    out_specs=[pl.BlockSpec((_RB, DH), lambda i: (i, 0)),
               pl.BlockSpec((_RB, DH), lambda i: (i, 0))],
    out_shape=[jax.ShapeDtypeStruct((N_NODES, DH), jnp.float32),
               jax.ShapeDtypeStruct((N_NODES, DH), jnp.float32)],
)

_final = pl.pallas_call(
    _final_body,
    grid=(_GRID,),
    in_specs=[
        pl.BlockSpec((NC, _RB, DH), lambda i: (0, i, 0)),
        pl.BlockSpec((NC, _RB, DH), lambda i: (0, i, 0)),
        pl.BlockSpec((_RB, DH), lambda i: (i, 0)),
        pl.BlockSpec((_RB, DH), lambda i: (i, 0)),
        pl.BlockSpec((_RB, 1), lambda i: (i, 0)),
        pl.BlockSpec((1, D_OUT), lambda i: (0, 0)),
    ],
    out_specs=pl.BlockSpec((_RB, D_OUT), lambda i: (i, 0)),
    out_shape=jax.ShapeDtypeStruct((N_NODES, D_OUT), jnp.float32),
)


@jax.jit
def _run(x, edge_index, W1, b1, W2, b2):
    e3 = edge_index.astype(jnp.int32).reshape(2, EROWS, CHUNK)

    dis, xs_lo, xs_hi = _deg_dis_xs(e3, x)
    dis_col = dis.reshape(N_PAD, 1)

    p1_lo, p1_hi = _scatter(xs_lo, xs_hi, e3)
    gs_lo, gs_hi = _mm(p1_lo, p1_hi, xs_lo, xs_hi, dis_col,
                       W1, b1.reshape(1, D_HID), W2)
    p2_lo, p2_hi = _scatter(gs_lo, gs_hi, e3)
    return _final(p2_lo, p2_hi, gs_lo, gs_hi, dis_col, b2.reshape(1, D_OUT))


def kernel(x, edge_index, edge_attr, W1, b1, W2, b2):
    return _run(x, edge_index, W1, b1, W2, b2)


# speedup vs baseline: 3.4479x; 1.1486x over previous
import jax
import jax.numpy as jnp
from jax import lax
from jax.experimental import pallas as pl
from jax.experimental.pallas import tpu as pltpu
from jax.experimental.pallas import tpu_sc as plsc

N_NODES = 10000
N_EDGES = 320000
D_IN = 128
D_HID = 256
D_OUT = 128
DH = D_IN // 2

NC = 2
NS = 16
L = 16

N_PAD = 10240
ROWS_PER_TILE = N_PAD // NS
CHUNK = 128
CHUNKS_PER_TILE = 80
EROWS = N_EDGES // CHUNK
MAIN_ROWS = 78
TAIL_ROW0 = NC * NS * MAIN_ROWS
DROWS = EROWS // NS

_mesh = plsc.VectorSubcoreMesh(core_axis_name="c", subcore_axis_name="s")
_sc_params = pltpu.CompilerParams(needs_layout_passes=False,
                                  use_tc_tiling_on_sc=False)


def _deg_dis_xs_body(e3_hbm, x_hbm, dis_hbm, xs_hbm,
                     dst_v, deg_v, stage, slice_v, dis_v, xbuf):
    c = lax.axis_index("c")
    s = lax.axis_index("s")

    pltpu.sync_copy(e3_hbm.at[1, pl.ds(s * DROWS, DROWS)],
                    dst_v.at[pl.ds(0, DROWS)])

    @pl.when(s < 4)
    def _():
        pltpu.sync_copy(e3_hbm.at[1, TAIL_ROW0 + s], dst_v.at[DROWS])

    def zero(i, _):
        deg_v[pl.ds(i * L, L)] = jnp.zeros((L,), jnp.float32)
        return 0
    lax.fori_loop(0, N_PAD // L, zero, 0)

    ones = jnp.ones((L,), jnp.float32)

    def count(i, _):
        idx = dst_v[i // (CHUNK // L), pl.ds((i % (CHUNK // L)) * L, L)]
        plsc.addupdate_scatter(deg_v, [idx], ones)
        return 0
    lax.fori_loop(0, DROWS * (CHUNK // L), count, 0)

    @pl.when(s < 4)
    def _():
        def count_tail(i, _):
            idx = dst_v[DROWS, pl.ds(i * L, L)]
            plsc.addupdate_scatter(deg_v, [idx], ones)
            return 0
        lax.fori_loop(0, CHUNK // L, count_tail, 0)

    pltpu.sync_copy(deg_v, stage.at[s])
    plsc.subcore_barrier()
    pltpu.sync_copy(stage.at[:, pl.ds(s * ROWS_PER_TILE, ROWS_PER_TILE)],
                    slice_v)

    def reduce_k(k, _):
        def add_row(r, a):
            return a + slice_v[r, pl.ds(k * L, L)]
        tot = lax.fori_loop(0, NS, add_row, jnp.zeros((L,), jnp.float32))
        d = tot + 1.0
        i32 = plsc.bitcast(d, jnp.int32)
        i32 = 0x5F3759DF - (i32 >> 1)
        y = plsc.bitcast(i32, jnp.float32)
        for _ in range(3):
            y = y * (1.5 - 0.5 * d * y * y)
        dis_v[pl.ds(k * L, L)] = y
        return 0
    lax.fori_loop(0, ROWS_PER_TILE // L, reduce_k, 0)

    @pl.when(c == 0)
    def _():
        pltpu.sync_copy(dis_v, dis_hbm.at[pl.ds(s * ROWS_PER_TILE,
                                                ROWS_PER_TILE)])

    half = ROWS_PER_TILE // NC
    base = s * ROWS_PER_TILE + c * half
    loc0 = c * half
    XB = 80

    def xs_chunk(ch, _):
        row0 = base + ch * XB

        @pl.when(row0 < N_NODES)
        def _():
            pltpu.sync_copy(x_hbm.at[pl.ds(row0, XB)], xbuf)

            def grp_fn(g, _):
                dvs = dis_v[pl.ds(loc0 + ch * XB + g * L, L)]
                for r in range(L):
                    dv = dvs[r]

                    def col_fn(j, _, r=r, dv=dv):
                        row = g * L + r
                        xbuf[row, pl.ds(j * L, L)] = (
                            xbuf[row, pl.ds(j * L, L)] * dv)
                        return 0
                    lax.fori_loop(0, D_IN // L, col_fn, 0)
                return 0
            lax.fori_loop(0, XB // L, grp_fn, 0)
            pltpu.sync_copy(xbuf, xs_hbm.at[pl.ds(row0, XB)])
        return 0
    lax.fori_loop(0, half // XB, xs_chunk, 0)


_deg_dis_xs = pl.kernel(
    _deg_dis_xs_body,
    out_type=(jax.ShapeDtypeStruct((N_PAD,), jnp.float32),
              jax.ShapeDtypeStruct((N_NODES, D_IN), jnp.float32)),
    mesh=_mesh,
    scratch_types=[
        pltpu.VMEM((DROWS + 1, CHUNK), jnp.int32),
        pltpu.VMEM((N_PAD,), jnp.float32),
        pltpu.VMEM_SHARED((NS, N_PAD), jnp.float32),
        pltpu.VMEM((NS, ROWS_PER_TILE), jnp.float32),
        pltpu.VMEM((ROWS_PER_TILE,), jnp.float32),
        pltpu.VMEM((80, D_IN), jnp.float32),
    ],
    compiler_params=_sc_params,
)


NBUF = 4


def _scatter_body(tab_hbm, e3_hbm, out_hbm,
                  sidx, sidx2, didx, bufs, zbuf, acc, gsems, ssems):
    c = lax.axis_index("c")
    s = lax.axis_index("s")
    wid = s * NC + c

    pltpu.sync_copy(e3_hbm.at[0, pl.ds(wid * MAIN_ROWS, MAIN_ROWS)],
                    sidx.at[pl.ds(0, MAIN_ROWS)])
    pltpu.sync_copy(e3_hbm.at[1, pl.ds(wid * MAIN_ROWS, MAIN_ROWS)],
                    didx.at[pl.ds(0, MAIN_ROWS)])
    trow = TAIL_ROW0 + wid // 8
    tcol = (wid % 8) * L
    pltpu.sync_copy(e3_hbm.at[0, trow, pl.ds(tcol, L)],
                    sidx.at[MAIN_ROWS, pl.ds(0, L)])
    pltpu.sync_copy(e3_hbm.at[1, trow, pl.ds(tcol, L)],
                    didx.at[MAIN_ROWS, pl.ds(0, L)])
    iota = lax.iota(jnp.int32, L)
    pad_src = iota * 8 + wid
    for k in range(1, CHUNK // L):
        sidx[MAIN_ROWS, pl.ds(k * L, L)] = pad_src
        didx[MAIN_ROWS, pl.ds(k * L, L)] = (
            N_NODES + iota * 8 + (wid + k) % 8)
    for k in range(CHUNK // L):
        sidx[MAIN_ROWS + 1, pl.ds(k * L, L)] = pad_src
        didx[MAIN_ROWS + 1, pl.ds(k * L, L)] = (
            N_NODES + 128 + iota * 8 + (wid + k) % 8)

    ZR = 320
    row0 = s * ROWS_PER_TILE

    def zero(i, _):
        r = i // (DH // L)
        j = i % (DH // L)
        zbuf[r, pl.ds(j * L, L)] = jnp.zeros((L,), jnp.float32)
        return 0
    lax.fori_loop(0, ZR * (DH // L), zero, 0)

    for f in range(2):
        def xform(i, _, f=f):
            r = i // (CHUNK // L)
            k = (i % (CHUNK // L)) * L
            sidx2[r, pl.ds(k, L)] = sidx[r, pl.ds(k, L)] * 2 + f
            return 0
        lax.fori_loop(0, CHUNKS_PER_TILE * (CHUNK // L), xform, 0)

        pltpu.sync_copy(zbuf, acc.at[pl.ds(row0, ZR)])
        pltpu.sync_copy(zbuf, acc.at[pl.ds(row0 + ZR, ZR)])
        plsc.subcore_barrier()

        for b in range(NBUF):
            pltpu.async_copy(tab_hbm.at[sidx2.at[b]], bufs.at[b], gsems.at[b])

        def step(i, _):
            j0 = i * NBUF
            for b in range(NBUF):
                pltpu.make_async_copy(tab_hbm.at[sidx2.at[j0 + b]],
                                      bufs.at[b], gsems.at[b]).wait()
                pltpu.async_copy(bufs.at[b], acc.at[didx.at[j0 + b]],
                                 ssems.at[b], add=True)
            for b in range(NBUF):
                pltpu.make_async_copy(bufs.at[b], acc.at[didx.at[j0 + b]],
                                      ssems.at[b]).wait()
                pltpu.async_copy(tab_hbm.at[sidx2.at[j0 + b + NBUF]],
                                 bufs.at[b], gsems.at[b])
            return 0
        lax.fori_loop(0, CHUNKS_PER_TILE // NBUF - 1, step, 0)

        j0 = CHUNKS_PER_TILE - NBUF
        for b in range(NBUF):
            pltpu.make_async_copy(tab_hbm.at[sidx2.at[j0 + b]],
                                  bufs.at[b], gsems.at[b]).wait()
            pltpu.async_copy(bufs.at[b], acc.at[didx.at[j0 + b]],
                             ssems.at[b], add=True)
        for b in range(NBUF):
            pltpu.make_async_copy(bufs.at[b], acc.at[didx.at[j0 + b]],
                                  ssems.at[b]).wait()

        plsc.subcore_barrier()
        pltpu.sync_copy(acc.at[pl.ds(row0, ROWS_PER_TILE)],
                        out_hbm.at[c, pl.ds(row0, ROWS_PER_TILE),
                                   pl.ds(f * DH, DH)])


_scatter = pl.kernel(
    _scatter_body,
    out_type=jax.ShapeDtypeStruct((NC, N_PAD, D_IN), jnp.float32),
    mesh=_mesh,
    scratch_types=[
        pltpu.VMEM((CHUNKS_PER_TILE, CHUNK), jnp.int32),
        pltpu.VMEM((CHUNKS_PER_TILE, CHUNK), jnp.int32),
        pltpu.VMEM((CHUNKS_PER_TILE, CHUNK), jnp.int32),
        pltpu.VMEM((NBUF, CHUNK, DH), jnp.float32),
        pltpu.VMEM((320, DH), jnp.float32),
        pltpu.VMEM_SHARED((N_PAD, DH), jnp.float32),
        pltpu.SemaphoreType.DMA((NBUF,)),
        pltpu.SemaphoreType.DMA((NBUF,)),
    ],
    compiler_params=_sc_params,
)


def _mm_body(p_ref, xs_ref, dis_ref, w1_ref, b1_ref, w2_ref, g_ref):
    a = (p_ref[0] + p_ref[1] + xs_ref[...]) * dis_ref[...]
    h = jnp.dot(a, w1_ref[...], preferred_element_type=jnp.float32)
    h = jnp.maximum(h + b1_ref[...], 0.0)
    g = jnp.dot(h, w2_ref[...], preferred_element_type=jnp.float32)
    g_ref[...] = g * dis_ref[...]


def _final_body(p_ref, gs_ref, dis_ref, b2_ref, out_ref):
    out_ref[...] = ((p_ref[0] + p_ref[1] + gs_ref[...]) * dis_ref[...]
                    + b2_ref[...])


_RB = 400
_GRID = N_NODES // _RB

_mm = pl.pallas_call(
    _mm_body,
    grid=(_GRID,),
    in_specs=[
        pl.BlockSpec((NC, _RB, D_IN), lambda i: (0, i, 0)),
        pl.BlockSpec((_RB, D_IN), lambda i: (i, 0)),
        pl.BlockSpec((_RB, 1), lambda i: (i, 0)),
        pl.BlockSpec((D_IN, D_HID), lambda i: (0, 0)),
        pl.BlockSpec((1, D_HID), lambda i: (0, 0)),
        pl.BlockSpec((D_HID, D_OUT), lambda i: (0, 0)),
    ],
    out_specs=pl.BlockSpec((_RB, D_OUT), lambda i: (i, 0)),
    out_shape=jax.ShapeDtypeStruct((N_NODES, D_OUT), jnp.float32),
)

_final = pl.pallas_call(
    _final_body,
    grid=(_GRID,),
    in_specs=[
        pl.BlockSpec((NC, _RB, D_OUT), lambda i: (0, i, 0)),
        pl.BlockSpec((_RB, D_OUT), lambda i: (i, 0)),
        pl.BlockSpec((_RB, 1), lambda i: (i, 0)),
        pl.BlockSpec((1, D_OUT), lambda i: (0, 0)),
    ],
    out_specs=pl.BlockSpec((_RB, D_OUT), lambda i: (i, 0)),
    out_shape=jax.ShapeDtypeStruct((N_NODES, D_OUT), jnp.float32),
)


@jax.jit
def _run(x, edge_index, W1, b1, W2, b2):
    e3 = edge_index.astype(jnp.int32).reshape(2, EROWS, CHUNK)

    dis, xs = _deg_dis_xs(e3, x)
    dis_col = dis.reshape(N_PAD, 1)

    p1 = _scatter(xs.reshape(2 * N_NODES, DH), e3)
    gs = _mm(p1, xs, dis_col, W1, b1.reshape(1, D_HID), W2)
    p2 = _scatter(gs.reshape(2 * N_NODES, DH), e3)
    return _final(p2, gs, dis_col, b2.reshape(1, D_OUT))


def kernel(x, edge_index, edge_attr, W1, b1, W2, b2):
    return _run(x, edge_index, W1, b1, W2, b2)


# speedup vs baseline: 3.7675x; 1.0927x over previous
import jax
import jax.numpy as jnp
from jax import lax
from jax.experimental import pallas as pl
from jax.experimental.pallas import tpu as pltpu
from jax.experimental.pallas import tpu_sc as plsc

N_NODES = 10000
N_EDGES = 320000
D_IN = 128
D_HID = 256
D_OUT = 128
DH = D_IN // 2

NC = 2
NS = 16
L = 16

N_PAD = 10240
ROWS_PER_TILE = N_PAD // NS
CHUNK = 128
CHUNKS_PER_TILE = 80
EROWS = N_EDGES // CHUNK
MAIN_ROWS = 78
TAIL_ROW0 = NC * NS * MAIN_ROWS
DROWS = EROWS // NS

_mesh = plsc.VectorSubcoreMesh(core_axis_name="c", subcore_axis_name="s")
_sc_params = pltpu.CompilerParams(needs_layout_passes=False,
                                  use_tc_tiling_on_sc=False)


def _deg_dis_xs_body(e3_hbm, x_hbm, dis_hbm, xs_hbm,
                     dst_v, deg_v, stage, slice_v, dis_v, xbuf):
    c = lax.axis_index("c")
    s = lax.axis_index("s")

    pltpu.sync_copy(e3_hbm.at[1, pl.ds(s * DROWS, DROWS)],
                    dst_v.at[pl.ds(0, DROWS)])

    @pl.when(s < 4)
    def _():
        pltpu.sync_copy(e3_hbm.at[1, TAIL_ROW0 + s], dst_v.at[DROWS])

    def zero(i, _):
        for u in range(4):
            deg_v[pl.ds((i * 4 + u) * L, L)] = jnp.zeros((L,), jnp.float32)
        return 0
    lax.fori_loop(0, N_PAD // L // 4, zero, 0)

    ones = jnp.ones((L,), jnp.float32)

    def count(r, _):
        for u in range(CHUNK // L):
            idx = dst_v[r, pl.ds(u * L, L)]
            plsc.addupdate_scatter(deg_v, [idx], ones)
        return 0
    lax.fori_loop(0, DROWS, count, 0)

    @pl.when(s < 4)
    def _():
        def count_tail(i, _):
            idx = dst_v[DROWS, pl.ds(i * L, L)]
            plsc.addupdate_scatter(deg_v, [idx], ones)
            return 0
        lax.fori_loop(0, CHUNK // L, count_tail, 0)

    pltpu.sync_copy(deg_v, stage.at[s])
    plsc.subcore_barrier()
    pltpu.sync_copy(stage.at[:, pl.ds(s * ROWS_PER_TILE, ROWS_PER_TILE)],
                    slice_v)

    def reduce_k(k, _):
        tot = slice_v[0, pl.ds(k * L, L)]
        for r in range(1, NS):
            tot = tot + slice_v[r, pl.ds(k * L, L)]
        d = tot + 1.0
        i32 = plsc.bitcast(d, jnp.int32)
        i32 = 0x5F3759DF - (i32 >> 1)
        y = plsc.bitcast(i32, jnp.float32)
        for _ in range(3):
            y = y * (1.5 - 0.5 * d * y * y)
        dis_v[pl.ds(k * L, L)] = y
        return 0
    lax.fori_loop(0, ROWS_PER_TILE // L, reduce_k, 0)

    @pl.when(c == 0)
    def _():
        pltpu.sync_copy(dis_v, dis_hbm.at[pl.ds(s * ROWS_PER_TILE,
                                                ROWS_PER_TILE)])

    half = ROWS_PER_TILE // NC
    base = s * ROWS_PER_TILE + c * half
    loc0 = c * half
    XB = 80

    def xs_chunk(ch, _):
        row0 = base + ch * XB

        @pl.when(row0 < N_NODES)
        def _():
            pltpu.sync_copy(x_hbm.at[pl.ds(row0, XB)], xbuf)

            def grp_fn(g, _):
                dvs = dis_v[pl.ds(loc0 + ch * XB + g * L, L)]
                for r in range(L):
                    dv = dvs[r]
                    row = g * L + r
                    for j in range(D_IN // L):
                        xbuf[row, pl.ds(j * L, L)] = (
                            xbuf[row, pl.ds(j * L, L)] * dv)
                return 0
            lax.fori_loop(0, XB // L, grp_fn, 0)
            pltpu.sync_copy(xbuf, xs_hbm.at[pl.ds(row0, XB)])
        return 0
    lax.fori_loop(0, half // XB, xs_chunk, 0)


_deg_dis_xs = pl.kernel(
    _deg_dis_xs_body,
    out_type=(jax.ShapeDtypeStruct((N_PAD,), jnp.float32),
              jax.ShapeDtypeStruct((N_NODES, D_IN), jnp.float32)),
    mesh=_mesh,
    scratch_types=[
        pltpu.VMEM((DROWS + 1, CHUNK), jnp.int32),
        pltpu.VMEM((N_PAD,), jnp.float32),
        pltpu.VMEM_SHARED((NS, N_PAD), jnp.float32),
        pltpu.VMEM((NS, ROWS_PER_TILE), jnp.float32),
        pltpu.VMEM((ROWS_PER_TILE,), jnp.float32),
        pltpu.VMEM((80, D_IN), jnp.float32),
    ],
    compiler_params=_sc_params,
)


NBUF = 4


def _scatter_body(tab_hbm, e3_hbm, out_hbm,
                  sidx, sidx2, didx, bufs, zbuf, acc, gsems, ssems):
    c = lax.axis_index("c")
    s = lax.axis_index("s")
    wid = s * NC + c

    pltpu.sync_copy(e3_hbm.at[0, pl.ds(wid * MAIN_ROWS, MAIN_ROWS)],
                    sidx.at[pl.ds(0, MAIN_ROWS)])
    pltpu.sync_copy(e3_hbm.at[1, pl.ds(wid * MAIN_ROWS, MAIN_ROWS)],
                    didx.at[pl.ds(0, MAIN_ROWS)])
    trow = TAIL_ROW0 + wid // 8
    tcol = (wid % 8) * L
    pltpu.sync_copy(e3_hbm.at[0, trow, pl.ds(tcol, L)],
                    sidx.at[MAIN_ROWS, pl.ds(0, L)])
    pltpu.sync_copy(e3_hbm.at[1, trow, pl.ds(tcol, L)],
                    didx.at[MAIN_ROWS, pl.ds(0, L)])
    iota = lax.iota(jnp.int32, L)
    pad_src = iota * 8 + wid
    for k in range(1, CHUNK // L):
        sidx[MAIN_ROWS, pl.ds(k * L, L)] = pad_src
        didx[MAIN_ROWS, pl.ds(k * L, L)] = (
            N_NODES + iota * 8 + (wid + k) % 8)
    for k in range(CHUNK // L):
        sidx[MAIN_ROWS + 1, pl.ds(k * L, L)] = pad_src
        didx[MAIN_ROWS + 1, pl.ds(k * L, L)] = (
            N_NODES + 128 + iota * 8 + (wid + k) % 8)

    ZR = 320
    row0 = s * ROWS_PER_TILE

    def zero(r, _):
        for j in range(DH // L):
            zbuf[r, pl.ds(j * L, L)] = jnp.zeros((L,), jnp.float32)
        return 0
    lax.fori_loop(0, ZR, zero, 0)

    for f in range(2):
        def xform(r, _, f=f):
            for k in range(CHUNK // L):
                sidx2[r, pl.ds(k * L, L)] = sidx[r, pl.ds(k * L, L)] * 2 + f
            return 0
        lax.fori_loop(0, CHUNKS_PER_TILE, xform, 0)

        pltpu.sync_copy(zbuf, acc.at[pl.ds(row0, ZR)])
        pltpu.sync_copy(zbuf, acc.at[pl.ds(row0 + ZR, ZR)])
        plsc.subcore_barrier()

        for b in range(NBUF):
            pltpu.async_copy(tab_hbm.at[sidx2.at[b]], bufs.at[b], gsems.at[b])

        def step(i, _):
            j0 = i * NBUF
            for b in range(NBUF):
                pltpu.make_async_copy(tab_hbm.at[sidx2.at[j0 + b]],
                                      bufs.at[b], gsems.at[b]).wait()
                pltpu.async_copy(bufs.at[b], acc.at[didx.at[j0 + b]],
                                 ssems.at[b], add=True)
            for b in range(NBUF):
                pltpu.make_async_copy(bufs.at[b], acc.at[didx.at[j0 + b]],
                                      ssems.at[b]).wait()
                pltpu.async_copy(tab_hbm.at[sidx2.at[j0 + b + NBUF]],
                                 bufs.at[b], gsems.at[b])
            return 0
        lax.fori_loop(0, CHUNKS_PER_TILE // NBUF - 1, step, 0)

        j0 = CHUNKS_PER_TILE - NBUF
        for b in range(NBUF):
            pltpu.make_async_copy(tab_hbm.at[sidx2.at[j0 + b]],
                                  bufs.at[b], gsems.at[b]).wait()
            pltpu.async_copy(bufs.at[b], acc.at[didx.at[j0 + b]],
                             ssems.at[b], add=True)
        for b in range(NBUF):
            pltpu.make_async_copy(bufs.at[b], acc.at[didx.at[j0 + b]],
                                  ssems.at[b]).wait()

        plsc.subcore_barrier()
        pltpu.sync_copy(acc.at[pl.ds(row0, ROWS_PER_TILE)],
                        out_hbm.at[c, pl.ds(row0, ROWS_PER_TILE),
                                   pl.ds(f * DH, DH)])


_scatter = pl.kernel(
    _scatter_body,
    out_type=jax.ShapeDtypeStruct((NC, N_PAD, D_IN), jnp.float32),
    mesh=_mesh,
    scratch_types=[
        pltpu.VMEM((CHUNKS_PER_TILE, CHUNK), jnp.int32),
        pltpu.VMEM((CHUNKS_PER_TILE, CHUNK), jnp.int32),
        pltpu.VMEM((CHUNKS_PER_TILE, CHUNK), jnp.int32),
        pltpu.VMEM((NBUF, CHUNK, DH), jnp.float32),
        pltpu.VMEM((320, DH), jnp.float32),
        pltpu.VMEM_SHARED((N_PAD, DH), jnp.float32),
        pltpu.SemaphoreType.DMA((NBUF,)),
        pltpu.SemaphoreType.DMA((NBUF,)),
    ],
    compiler_params=_sc_params,
)


def _mm_body(p_ref, xs_ref, dis_ref, w1_ref, b1_ref, w2_ref, g_ref):
    a = (p_ref[0] + p_ref[1] + xs_ref[...]) * dis_ref[...]
    h = jnp.dot(a, w1_ref[...], preferred_element_type=jnp.float32)
    h = jnp.maximum(h + b1_ref[...], 0.0)
    g = jnp.dot(h, w2_ref[...], preferred_element_type=jnp.float32)
    g_ref[...] = g * dis_ref[...]


def _final_body(p_ref, gs_ref, dis_ref, b2_ref, out_ref):
    out_ref[...] = ((p_ref[0] + p_ref[1] + gs_ref[...]) * dis_ref[...]
                    + b2_ref[...])


_RB = 1000
_GRID = N_NODES // _RB

_mm = pl.pallas_call(
    _mm_body,
    grid=(_GRID,),
    in_specs=[
        pl.BlockSpec((NC, _RB, D_IN), lambda i: (0, i, 0)),
        pl.BlockSpec((_RB, D_IN), lambda i: (i, 0)),
        pl.BlockSpec((_RB, 1), lambda i: (i, 0)),
        pl.BlockSpec((D_IN, D_HID), lambda i: (0, 0)),
        pl.BlockSpec((1, D_HID), lambda i: (0, 0)),
        pl.BlockSpec((D_HID, D_OUT), lambda i: (0, 0)),
    ],
    out_specs=pl.BlockSpec((_RB, D_OUT), lambda i: (i, 0)),
    out_shape=jax.ShapeDtypeStruct((N_NODES, D_OUT), jnp.float32),
)

_final = pl.pallas_call(
    _final_body,
    grid=(_GRID,),
    in_specs=[
        pl.BlockSpec((NC, _RB, D_OUT), lambda i: (0, i, 0)),
        pl.BlockSpec((_RB, D_OUT), lambda i: (i, 0)),
        pl.BlockSpec((_RB, 1), lambda i: (i, 0)),
        pl.BlockSpec((1, D_OUT), lambda i: (0, 0)),
    ],
    out_specs=pl.BlockSpec((_RB, D_OUT), lambda i: (i, 0)),
    out_shape=jax.ShapeDtypeStruct((N_NODES, D_OUT), jnp.float32),
)


@jax.jit
def _run(x, edge_index, W1, b1, W2, b2):
    e3 = edge_index.astype(jnp.int32).reshape(2, EROWS, CHUNK)

    dis, xs = _deg_dis_xs(e3, x)
    dis_col = dis.reshape(N_PAD, 1)

    p1 = _scatter(xs.reshape(2 * N_NODES, DH), e3)
    gs = _mm(p1, xs, dis_col, W1, b1.reshape(1, D_HID), W2)
    p2 = _scatter(gs.reshape(2 * N_NODES, DH), e3)
    return _final(p2, gs, dis_col, b2.reshape(1, D_OUT))


def kernel(x, edge_index, edge_attr, W1, b1, W2, b2):
    return _run(x, edge_index, W1, b1, W2, b2)
